# baseline (speedup 1.0000x reference)
import functools
import jax
import jax.numpy as jnp
from jax import lax
from jax.experimental import pallas as pl
import jax.experimental.pallas.tpu as pltpu
from jax.experimental.pallas import tpu_sc as plsc

_B = 16
_C = 768
_P = 512
_HW = 1024
_HALF = _HW // 2
_CR = 16
_NCH = _HALF // _CR
_K = 4
_W = 2
_D = _C + _P


def _sc_body(x_hbm, row_hbm, col_hbm, o_hbm, colbuf, rowbuf, stage,
             in_sems, out_sems):
    cid = lax.axis_index("c")
    sid = lax.axis_index("s")
    wid = sid * 2 + cid
    b = wid // 2
    half = wid % 2
    base = half * _HALF

    pltpu.sync_copy(col_hbm, colbuf)
    pltpu.sync_copy(row_hbm, rowbuf)

    for k in range(_K):
        wlo = (k % 2) * _CR
        for r in range(_CR):
            for g in range(16):
                stage[k, r, pl.ds(_C + g * 16, 16)] = colbuf[wlo + r, pl.ds(g * 16, 16)]

    def in_copy(i, k):
        return pltpu.make_async_copy(
            x_hbm.at[b, pl.ds(base + i * _CR, _CR)],
            stage.at[k, :, pl.ds(0, _C)], in_sems.at[k])

    def out_copy(i, k):
        return pltpu.make_async_copy(
            stage.at[k],
            o_hbm.at[b, pl.ds(base + i * _CR, _CR)], out_sems.at[k])

    for i in range(_K):
        in_copy(i, i % _K).start()

    def step(i, carry):
        k = lax.rem(i, _K)
        in_copy(i, k).wait()
        h = (base + i * _CR) // 32
        for g in range(16):
            v = rowbuf[h, pl.ds(g * 16, 16)]
            for r in range(_CR):
                stage[k, r, pl.ds(_C + 256 + g * 16, 16)] = v
        out_copy(i, k).start()
        j = i - _W

        @pl.when(jnp.logical_and(j >= 0, j + _K < _NCH))
        def _():
            out_copy(j, lax.rem(j, _K)).wait()
            in_copy(j + _K, lax.rem(j, _K)).start()

        return carry

    lax.fori_loop(0, _NCH, step, 0)
    for i in range(_NCH - _K, _NCH):
        out_copy(i, i % _K).wait()


def kernel(x, row_embed, col_embed):
    bsz, c, h, w = x.shape
    xt = x.transpose(0, 2, 3, 1).reshape(bsz, h * w, c)
    mesh = plsc.VectorSubcoreMesh(core_axis_name="c", subcore_axis_name="s")
    run = pl.kernel(
        _sc_body,
        out_type=jax.ShapeDtypeStruct((bsz, h * w, _D), x.dtype),
        mesh=mesh,
        scratch_types=[
            pltpu.VMEM((32, 256), x.dtype),
            pltpu.VMEM((32, 256), x.dtype),
            pltpu.VMEM((_K, _CR, _D), x.dtype),
            pltpu.SemaphoreType.DMA((_K,)),
            pltpu.SemaphoreType.DMA((_K,)),
        ],
    )
    out = run(xt, row_embed, col_embed)
    return out.reshape(bsz, h, w, _D).transpose(0, 3, 1, 2)
